# baseline (speedup 1.0000x reference)
import jax
import jax.numpy as jnp
from jax.experimental import pallas as pl
from jax.experimental.pallas import tpu as pltpu

_ROW_BLOCK = 400


def _linear_relu_kernel(x_ref, wt_ref, b_ref, o_ref):
    acc = jnp.dot(x_ref[...], wt_ref[...], preferred_element_type=jnp.float32)
    o_ref[...] = jnp.maximum(acc + b_ref[...], 0.0)


def kernel(feats, edge_index, W, b, agg_weight):
    del edge_index, agg_weight
    n, in_feats = feats.shape
    out_feats = W.shape[0]
    wt = W.T
    b2 = b.reshape(1, out_feats)
    grid = (pl.cdiv(n, _ROW_BLOCK),)
    return pl.pallas_call(
        _linear_relu_kernel,
        grid=grid,
        in_specs=[
            pl.BlockSpec((_ROW_BLOCK, in_feats), lambda i: (i, 0)),
            pl.BlockSpec((in_feats, out_feats), lambda i: (0, 0)),
            pl.BlockSpec((1, out_feats), lambda i: (0, 0)),
        ],
        out_specs=pl.BlockSpec((_ROW_BLOCK, out_feats), lambda i: (i, 0)),
        out_shape=jax.ShapeDtypeStruct((n, out_feats), jnp.float32),
        compiler_params=pltpu.CompilerParams(dimension_semantics=("parallel",)),
    )(feats, wt, b2)


# speedup vs baseline: 2.1115x; 2.1115x over previous
import jax
import jax.numpy as jnp
from jax.experimental import pallas as pl
from jax.experimental.pallas import tpu as pltpu

_ROW_BLOCK = 2000


def _linear_relu_kernel(x_ref, wt_ref, b_ref, o_ref):
    acc = jnp.dot(x_ref[...], wt_ref[...], preferred_element_type=jnp.float32)
    o_ref[...] = jnp.maximum(acc + b_ref[...], 0.0)


def kernel(feats, edge_index, W, b, agg_weight):
    del edge_index, agg_weight
    n, in_feats = feats.shape
    out_feats = W.shape[0]
    wt = W.T
    b2 = b.reshape(1, out_feats)
    grid = (pl.cdiv(n, _ROW_BLOCK),)
    return pl.pallas_call(
        _linear_relu_kernel,
        grid=grid,
        in_specs=[
            pl.BlockSpec((_ROW_BLOCK, in_feats), lambda i: (i, 0)),
            pl.BlockSpec((in_feats, out_feats), lambda i: (0, 0)),
            pl.BlockSpec((1, out_feats), lambda i: (0, 0)),
        ],
        out_specs=pl.BlockSpec((_ROW_BLOCK, out_feats), lambda i: (i, 0)),
        out_shape=jax.ShapeDtypeStruct((n, out_feats), jnp.float32),
        compiler_params=pltpu.CompilerParams(dimension_semantics=("parallel",)),
    )(feats, wt, b2)
